# Initial kernel scaffold; baseline (speedup 1.0000x reference)
import functools

import jax
import jax.numpy as jnp
from jax import lax
from jax.experimental import pallas as pl
from jax.experimental.pallas import tpu as pltpu
from jax.experimental.pallas import tpu_sc as plsc

D = 1024
NT = 4 * 2048
NW = 32
TPW = NT // NW
T = 16
NCHUNK = TPW // T
LANES = 16


def _spatial_body(c_hbm, w0, w1, w2, w3, w4, w5, out_hbm,
                  idx_v, r0, r1, r2, r3, r4, r5, sem):
    tabs = (w0, w1, w2, w3, w4, w5)
    rows = (r0, r1, r2, r3, r4, r5)
    wid = lax.axis_index("s") * 2 + lax.axis_index("c")
    base = wid * TPW

    def chunk_body(ci, carry):
        tok0 = base + ci * T
        pltpu.sync_copy(c_hbm.at[:, pl.ds(tok0, T)], idx_v)
        cps = [pltpu.async_copy(tabs[j].at[idx_v.at[j]], rows[j], sem)
               for j in range(6)]
        for cp in cps:
            cp.wait()

        def tok_body(t, carry2):
            def elem_body(e, carry3):
                s = pl.ds(e * LANES, LANES)
                acc = ((r0[t, s] + r1[t, s]) + (r2[t, s] + r3[t, s])
                       + (r4[t, s] + r5[t, s]))
                r0[t, s] = acc
                return carry3
            return lax.fori_loop(0, D // LANES, elem_body, carry2)

        lax.fori_loop(0, T, tok_body, carry)
        pltpu.sync_copy(r0, out_hbm.at[pl.ds(tok0, T)])
        return carry

    lax.fori_loop(0, NCHUNK, chunk_body, 0)


_spatial = functools.partial(
    pl.kernel,
    mesh=plsc.VectorSubcoreMesh(core_axis_name="c", subcore_axis_name="s"),
    out_type=jax.ShapeDtypeStruct((NT, D), jnp.float32),
    scratch_types=[pltpu.VMEM((6, T), jnp.int32)]
                  + [pltpu.VMEM((T, D), jnp.float32) for _ in range(6)]
                  + [pltpu.SemaphoreType.DMA],
)(_spatial_body)


def kernel(coordinates, W_tlx, W_tly, W_brx, W_bry, W_w, W_h):
    b, s, _ = coordinates.shape
    coords = coordinates.astype(jnp.int32).reshape(NT, 6).T
    out = _spatial(coords, W_tlx, W_tly, W_brx, W_bry, W_w, W_h)
    return out.reshape(b, s, D)


# speedup vs baseline: 1.2447x; 1.2447x over previous
import functools

import jax
import jax.numpy as jnp
from jax import lax
from jax.experimental import pallas as pl
from jax.experimental.pallas import tpu as pltpu
from jax.experimental.pallas import tpu_sc as plsc

D = 1024
NT = 4 * 2048
NW = 32
TPW = NT // NW
T = 16
NCHUNK = TPW // T
LANES = 16


def _spatial_body(c_hbm, w0, w1, w2, w3, w4, w5, out_hbm,
                  idx_v, r0, r1, r2, r3, r4, r5, sem):
    tabs = (w0, w1, w2, w3, w4, w5)
    rows = (r0, r1, r2, r3, r4, r5)
    wid = lax.axis_index("s") * 2 + lax.axis_index("c")
    base = wid * TPW

    def chunk_body(ci, carry):
        tok0 = base + ci * T
        for j in range(6):
            pltpu.sync_copy(c_hbm.at[j, pl.ds(tok0, T)], idx_v.at[j])
        cps = [pltpu.async_copy(tabs[j].at[idx_v.at[j]], rows[j], sem)
               for j in range(6)]
        for cp in cps:
            cp.wait()

        def tok_body(t, carry2):
            def elem_body(e, carry3):
                s = pl.ds(e * LANES, LANES)
                acc = ((r0[t, s] + r1[t, s]) + (r2[t, s] + r3[t, s])
                       + (r4[t, s] + r5[t, s]))
                r0[t, s] = acc
                return carry3
            return lax.fori_loop(0, D // LANES, elem_body, carry2)

        lax.fori_loop(0, T, tok_body, carry)
        pltpu.sync_copy(r0, out_hbm.at[pl.ds(tok0, T)])
        return carry

    lax.fori_loop(0, NCHUNK, chunk_body, 0)


_spatial = functools.partial(
    pl.kernel,
    mesh=plsc.VectorSubcoreMesh(core_axis_name="c", subcore_axis_name="s"),
    out_type=jax.ShapeDtypeStruct((NT, D), jnp.float32),
    scratch_types=[pltpu.VMEM((6, T), jnp.int32)]
                  + [pltpu.VMEM((T, D), jnp.float32) for _ in range(6)]
                  + [pltpu.SemaphoreType.DMA],
)(_spatial_body)


def kernel(coordinates, W_tlx, W_tly, W_brx, W_bry, W_w, W_h):
    b, s, _ = coordinates.shape
    coords = coordinates.astype(jnp.int32).reshape(NT, 6).T
    out = _spatial(coords, W_tlx, W_tly, W_brx, W_bry, W_w, W_h)
    return out.reshape(b, s, D)


# speedup vs baseline: 2.9629x; 2.3805x over previous
import functools

import jax
import jax.numpy as jnp
from jax import lax
from jax.experimental import pallas as pl
from jax.experimental.pallas import tpu as pltpu
from jax.experimental.pallas import tpu_sc as plsc

D = 1024
NT = 4 * 2048
NW = 32
TPW = NT // NW
T = 8
NCHUNK = TPW // T
LANES = 16


def _spatial_body(c_hbm, w0, w1, w2, w3, w4, w5, out_hbm,
                  idx_v, ra0, ra1, ra2, ra3, ra4, ra5,
                  rb0, rb1, rb2, rb3, rb4, rb5, oa, ob,
                  ga, gb, soa, sob):
    tabs = (w0, w1, w2, w3, w4, w5)
    rows = ((ra0, ra1, ra2, ra3, ra4, ra5),
            (rb0, rb1, rb2, rb3, rb4, rb5))
    outs = (oa, ob)
    gsems = (ga, gb)
    osems = (soa, sob)
    wid = lax.axis_index("s") * 2 + lax.axis_index("c")
    base = wid * TPW

    for j in range(6):
        pltpu.sync_copy(c_hbm.at[j, pl.ds(base, TPW)], idx_v.at[j])

    def gather_start(c, s):
        for j in range(6):
            pltpu.async_copy(tabs[j].at[idx_v.at[j, pl.ds(c * T, T)]],
                             rows[s][j], gsems[s])

    def gather_wait(s):
        for j in range(6):
            pltpu.make_async_copy(tabs[j].at[idx_v.at[j, pl.ds(0, T)]],
                                  rows[s][j], gsems[s]).wait()

    def combine_store(c, s):
        r0, r1, r2, r3, r4, r5 = rows[s]
        o = outs[s]

        def tok_body(t, carry):
            def elem_body(e, carry2):
                sl = pl.ds(e * LANES, LANES)
                o[t, sl] = ((r0[t, sl] + r1[t, sl]) + (r2[t, sl] + r3[t, sl])
                            + (r4[t, sl] + r5[t, sl]))
                return carry2
            return lax.fori_loop(0, D // LANES, elem_body, carry, unroll=8)

        lax.fori_loop(0, T, tok_body, 0)
        pltpu.async_copy(o, out_hbm.at[pl.ds(base + c * T, T)], osems[s])

    def out_wait(s):
        pltpu.make_async_copy(outs[s], out_hbm.at[pl.ds(base, T)],
                              osems[s]).wait()

    gather_start(0, 0)
    gather_start(1, 1)
    gather_wait(0)
    combine_store(0, 0)
    gather_start(2, 0)
    gather_wait(1)
    combine_store(1, 1)
    gather_start(3, 1)

    def pair_body(k, carry):
        c0 = k * 2
        gather_wait(0)
        out_wait(0)
        combine_store(c0, 0)
        gather_start(c0 + 2, 0)
        gather_wait(1)
        out_wait(1)
        combine_store(c0 + 1, 1)
        gather_start(c0 + 3, 1)
        return carry

    lax.fori_loop(1, NCHUNK // 2 - 1, pair_body, 0)

    gather_wait(0)
    out_wait(0)
    combine_store(NCHUNK - 2, 0)
    gather_wait(1)
    out_wait(1)
    combine_store(NCHUNK - 1, 1)
    out_wait(0)
    out_wait(1)


_spatial = functools.partial(
    pl.kernel,
    mesh=plsc.VectorSubcoreMesh(core_axis_name="c", subcore_axis_name="s"),
    out_type=jax.ShapeDtypeStruct((NT, D), jnp.float32),
    scratch_types=[pltpu.VMEM((6, TPW), jnp.int32)]
                  + [pltpu.VMEM((T, D), jnp.float32) for _ in range(14)]
                  + [pltpu.SemaphoreType.DMA for _ in range(4)],
)(_spatial_body)


def kernel(coordinates, W_tlx, W_tly, W_brx, W_bry, W_w, W_h):
    b, s, _ = coordinates.shape
    coords = coordinates.astype(jnp.int32).reshape(NT, 6).T
    out = _spatial(coords, W_tlx, W_tly, W_brx, W_bry, W_w, W_h)
    return out.reshape(b, s, D)
